# baseline (speedup 1.0000x reference)
import functools

import jax
import jax.numpy as jnp
from jax import lax
from jax.experimental import pallas as pl
from jax.experimental.pallas import tpu as pltpu
from jax.experimental.pallas import tpu_sc as plsc

_NS = 5
_ROWS = 16384
_D = 64
_NTILES = 32
_RPT = _ROWS // _NTILES
_NGRP = _RPT // 16

_LOG_EPS = -20.72326583694641
_LOG_BETA = -0.2876820724517809
_LN2 = 0.6931471805599453
_SQRT2 = 1.4142135623730951


def _vlog(x):
    b = plsc.bitcast(x, jnp.int32)
    e = (b >> 23) - 127
    m = plsc.bitcast((b & 0x7FFFFF) | 0x3F800000, jnp.float32)
    big = m > _SQRT2
    m = jnp.where(big, m * 0.5, m)
    e = (e + jnp.where(big, 1, 0)).astype(jnp.float32)
    z = m - 1.0
    p = jnp.float32(7.0376836292e-2)
    p = p * z - 1.1514610310e-1
    p = p * z + 1.1676998740e-1
    p = p * z - 1.2420140846e-1
    p = p * z + 1.4249322787e-1
    p = p * z - 1.6668057665e-1
    p = p * z + 2.0000714765e-1
    p = p * z - 2.4999993993e-1
    p = p * z + 3.3333331174e-1
    z2 = z * z
    return z + z * z2 * p - 0.5 * z2 + e * _LN2


def _logsig_floor(x, floor):
    return jnp.maximum(-_vlog(1.0 + jnp.exp(x)), floor)


def _tc_gather_body(idx_ref, emb_ref, out_ref, etmp, sem):
    etmp[...] = jnp.zeros_like(etmp)
    for s in range(_NS):
        cp = pltpu.make_async_copy(
            emb_ref.at[pl.ds(idx_ref[s], 1)], etmp.at[pl.ds(s, 1)], sem)
        cp.start()
        cp.wait()
    ev = etmp[...]
    out_ref[...] = jnp.concatenate([ev, jnp.zeros_like(ev)], axis=1)


def _tc_gather(emb_table, sample_indices):
    grid_spec = pltpu.PrefetchScalarGridSpec(
        num_scalar_prefetch=1,
        grid=(1,),
        in_specs=[pl.BlockSpec(memory_space=pltpu.MemorySpace.HBM)],
        out_specs=pl.BlockSpec((8, 2 * _D), lambda i, idx: (0, 0)),
        scratch_shapes=[
            pltpu.VMEM((8, _D), jnp.float32),
            pltpu.SemaphoreType.DMA,
        ],
    )
    return pl.pallas_call(
        _tc_gather_body,
        grid_spec=grid_spec,
        out_shape=jax.ShapeDtypeStruct((8, 2 * _D), jnp.float32),
    )(sample_indices, emb_table)


_CH = 128
_NCHUNK = _RPT // _CH
_GPC = _CH // 16


def _sc_body(ctx_hbm, tgt_hbm, erows_hbm, out_hbm,
             cbuf0, cbuf1, tbuf0, tbuf1, erows, resb,
             sem_c0, sem_c1, sem_t0, sem_t1, sem_e):
    cid = lax.axis_index("c")
    sid = lax.axis_index("s")
    wid = sid * 2 + cid
    base = wid * _RPT

    pltpu.async_copy(erows_hbm, erows, sem_e).wait()

    iota = lax.iota(jnp.int32, 16)
    srow = [jnp.full((16,), s, jnp.int32) for s in range(_NS)]

    def chunk_total(cb, tb, total):
        def group(g, tot):
            ridx = g * 16 + iota
            acc_t = jnp.zeros((16,), jnp.float32)
            accs = [jnp.zeros((16,), jnp.float32) for _ in range(_NS)]
            for k in range(_D):
                cidx = jnp.full((16,), k, jnp.int32)
                c_col = plsc.load_gather(cb, [ridx, cidx])
                t_col = plsc.load_gather(tb, [ridx, cidx])
                acc_t = acc_t + c_col * t_col
                for s in range(_NS):
                    e_b = plsc.load_gather(erows, [srow[s], cidx])
                    accs[s] = accs[s] + c_col * e_b
            term = _logsig_floor(-acc_t, _LOG_EPS)
            for s in range(_NS):
                term = term + _logsig_floor(accs[s], _LOG_BETA)
            return tot + term

        return lax.fori_loop(0, _GPC, group, total)

    bufs = [(cbuf0, tbuf0, sem_c0, sem_t0), (cbuf1, tbuf1, sem_c1, sem_t1)]

    def start(i):
        cb, tb, sc_, st_ = bufs[i % 2]
        hc = pltpu.async_copy(ctx_hbm.at[pl.ds(base + i * _CH, _CH)], cb, sc_)
        ht = pltpu.async_copy(tgt_hbm.at[pl.ds(base + i * _CH, _CH)], tb, st_)
        return hc, ht

    handles = {0: start(0)}
    total = jnp.zeros((16,), jnp.float32)
    for i in range(_NCHUNK):
        if i + 1 < _NCHUNK:
            handles[i + 1] = start(i + 1)
        hc, ht = handles.pop(i)
        hc.wait()
        ht.wait()
        cb, tb, _, _ = bufs[i % 2]
        total = chunk_total(cb, tb, total)

    resb[...] = total
    pltpu.sync_copy(resb, out_hbm.at[pl.ds(wid * 16, 16)])


def kernel(context, target, emb_table, sample_indices):
    mesh = plsc.VectorSubcoreMesh(core_axis_name="c", subcore_axis_name="s")
    sc = functools.partial(
        pl.kernel,
        out_type=jax.ShapeDtypeStruct((512,), jnp.float32),
        mesh=mesh,
        compiler_params=pltpu.CompilerParams(needs_layout_passes=False),
        scratch_types=[
            pltpu.VMEM((_CH, _D), jnp.float32),
            pltpu.VMEM((_CH, _D), jnp.float32),
            pltpu.VMEM((_CH, _D), jnp.float32),
            pltpu.VMEM((_CH, _D), jnp.float32),
            pltpu.VMEM((8, 2 * _D), jnp.float32),
            pltpu.VMEM((16,), jnp.float32),
            pltpu.SemaphoreType.DMA,
            pltpu.SemaphoreType.DMA,
            pltpu.SemaphoreType.DMA,
            pltpu.SemaphoreType.DMA,
            pltpu.SemaphoreType.DMA,
        ],
    )(_sc_body)
    erows = _tc_gather(emb_table, sample_indices.astype(jnp.int32))
    out = sc(context, target, erows)
    return jnp.sum(out)


# speedup vs baseline: 1.1366x; 1.1366x over previous
import functools

import jax
import jax.numpy as jnp
from jax import lax
from jax.experimental import pallas as pl
from jax.experimental.pallas import tpu as pltpu
from jax.experimental.pallas import tpu_sc as plsc

_NS = 5
_ROWS = 16384
_D = 64
_NTILES = 32
_RPT = _ROWS // _NTILES
_NGRP = _RPT // 16

_LOG_EPS = -20.72326583694641
_LOG_BETA = -0.2876820724517809
_LN2 = 0.6931471805599453
_SQRT2 = 1.4142135623730951


def _vlog(x):
    b = plsc.bitcast(x, jnp.int32)
    e = (b >> 23) - 127
    m = plsc.bitcast((b & 0x7FFFFF) | 0x3F800000, jnp.float32)
    big = m > _SQRT2
    m = jnp.where(big, m * 0.5, m)
    e = (e + jnp.where(big, 1, 0)).astype(jnp.float32)
    z = m - 1.0
    p = jnp.float32(7.0376836292e-2)
    p = p * z - 1.1514610310e-1
    p = p * z + 1.1676998740e-1
    p = p * z - 1.2420140846e-1
    p = p * z + 1.4249322787e-1
    p = p * z - 1.6668057665e-1
    p = p * z + 2.0000714765e-1
    p = p * z - 2.4999993993e-1
    p = p * z + 3.3333331174e-1
    z2 = z * z
    return z + z * z2 * p - 0.5 * z2 + e * _LN2


def _logsig_floor(x, floor):
    return jnp.maximum(-_vlog(1.0 + jnp.exp(x)), floor)


def _tc_gather_body(idx_ref, emb_ref, out_ref, etmp, sem):
    etmp[...] = jnp.zeros_like(etmp)
    for s in range(_NS):
        cp = pltpu.make_async_copy(
            emb_ref.at[pl.ds(idx_ref[s], 1)], etmp.at[pl.ds(s, 1)], sem)
        cp.start()
        cp.wait()
    ev = etmp[...]
    out_ref[...] = jnp.concatenate([ev, jnp.zeros_like(ev)], axis=1)


def _tc_gather(emb_table, sample_indices):
    grid_spec = pltpu.PrefetchScalarGridSpec(
        num_scalar_prefetch=1,
        grid=(1,),
        in_specs=[pl.BlockSpec(memory_space=pltpu.MemorySpace.HBM)],
        out_specs=pl.BlockSpec((8, 2 * _D), lambda i, idx: (0, 0)),
        scratch_shapes=[
            pltpu.VMEM((8, _D), jnp.float32),
            pltpu.SemaphoreType.DMA,
        ],
    )
    return pl.pallas_call(
        _tc_gather_body,
        grid_spec=grid_spec,
        out_shape=jax.ShapeDtypeStruct((8, 2 * _D), jnp.float32),
    )(sample_indices, emb_table)


_CH = 128
_NCHUNK = _RPT // _CH
_GPC = _CH // 16


def _sc_body(ctx_hbm, tgt_hbm, erows_hbm, out_hbm,
             cbuf0, cbuf1, tbuf0, tbuf1, erows, resb,
             sem_c0, sem_c1, sem_t0, sem_t1, sem_e):
    cid = lax.axis_index("c")
    sid = lax.axis_index("s")
    wid = sid * 2 + cid
    base = wid * _RPT

    pltpu.async_copy(erows_hbm, erows, sem_e).wait()

    iota = lax.iota(jnp.int32, 16)
    evecs = [[erows[s, pl.ds(16 * j, 16)] for j in range(_D // 16)]
             for s in range(_NS)]

    def chunk_total(cb, tb, total):
        def group(g, tot):
            ridx = g * 16 + iota
            acc_t = jnp.zeros((16,), jnp.float32)
            accs = [jnp.zeros((16,), jnp.float32) for _ in range(_NS)]
            for k in range(_D):
                cidx = jnp.full((16,), k, jnp.int32)
                c_col = plsc.load_gather(cb, [ridx, cidx])
                t_col = plsc.load_gather(tb, [ridx, cidx])
                acc_t = acc_t + c_col * t_col
                for s in range(_NS):
                    accs[s] = accs[s] + c_col * evecs[s][k // 16][k % 16]
            term = _logsig_floor(-acc_t, _LOG_EPS)
            for s in range(_NS):
                term = term + _logsig_floor(accs[s], _LOG_BETA)
            return tot + term

        return lax.fori_loop(0, _GPC, group, total)

    bufs = [(cbuf0, tbuf0, sem_c0, sem_t0), (cbuf1, tbuf1, sem_c1, sem_t1)]

    def start(i):
        cb, tb, sc_, st_ = bufs[i % 2]
        hc = pltpu.async_copy(ctx_hbm.at[pl.ds(base + i * _CH, _CH)], cb, sc_)
        ht = pltpu.async_copy(tgt_hbm.at[pl.ds(base + i * _CH, _CH)], tb, st_)
        return hc, ht

    handles = {0: start(0)}
    total = jnp.zeros((16,), jnp.float32)
    for i in range(_NCHUNK):
        if i + 1 < _NCHUNK:
            handles[i + 1] = start(i + 1)
        hc, ht = handles.pop(i)
        hc.wait()
        ht.wait()
        cb, tb, _, _ = bufs[i % 2]
        total = chunk_total(cb, tb, total)

    resb[...] = total
    pltpu.sync_copy(resb, out_hbm.at[pl.ds(wid * 16, 16)])


def kernel(context, target, emb_table, sample_indices):
    mesh = plsc.VectorSubcoreMesh(core_axis_name="c", subcore_axis_name="s")
    sc = functools.partial(
        pl.kernel,
        out_type=jax.ShapeDtypeStruct((512,), jnp.float32),
        mesh=mesh,
        compiler_params=pltpu.CompilerParams(needs_layout_passes=False),
        scratch_types=[
            pltpu.VMEM((_CH, _D), jnp.float32),
            pltpu.VMEM((_CH, _D), jnp.float32),
            pltpu.VMEM((_CH, _D), jnp.float32),
            pltpu.VMEM((_CH, _D), jnp.float32),
            pltpu.VMEM((8, 2 * _D), jnp.float32),
            pltpu.VMEM((16,), jnp.float32),
            pltpu.SemaphoreType.DMA,
            pltpu.SemaphoreType.DMA,
            pltpu.SemaphoreType.DMA,
            pltpu.SemaphoreType.DMA,
            pltpu.SemaphoreType.DMA,
        ],
    )(_sc_body)
    erows = _tc_gather(emb_table, sample_indices.astype(jnp.int32))
    out = sc(context, target, erows)
    return jnp.sum(out)
